# Initial kernel scaffold; baseline (speedup 1.0000x reference)
import jax
import jax.numpy as jnp
from jax.experimental import pallas as pl


def kernel(H, row, col, val, W):
    raise NotImplementedError("write your pallas kernel here")


# speedup vs baseline: 11.3453x; 11.3453x over previous
import functools

import jax
import jax.numpy as jnp
from jax import lax
from jax.experimental import pallas as pl
from jax.experimental.pallas import tpu as pltpu
from jax.experimental.pallas import tpu_sc as plsc

NC = 2
NS = 16
NW = NC * NS
LANES = 16
CHUNK = 80


def _sc_spmm(N, E, D):
    assert E % NW == 0 and D % LANES == 0
    epw = E // NW
    nchunk = epw // CHUNK
    assert nchunk * CHUNK == epw and nchunk % 2 == 1
    npair = nchunk // 2
    rows_pt = (N // NS) // 8 * 8
    tail = N - NS * rows_pt
    assert tail % 8 == 0 and tail <= CHUNK

    mesh = plsc.VectorSubcoreMesh(core_axis_name="c", subcore_axis_name="s",
                                  num_cores=NC, num_subcores=NS)

    @functools.partial(
        pl.kernel,
        out_type=jax.ShapeDtypeStruct((NC, N, D), jnp.float32),
        mesh=mesh,
        scratch_types=[
            pltpu.VMEM((epw,), jnp.int32),
            pltpu.VMEM((2, CHUNK), jnp.int32),
            pltpu.VMEM((2, CHUNK), jnp.float32),
            pltpu.VMEM((CHUNK, D), jnp.float32),
            pltpu.VMEM((CHUNK, D), jnp.float32),
            pltpu.VMEM_SHARED((N, D), jnp.float32),
            pltpu.SemaphoreType.DMA,
            pltpu.SemaphoreType.DMA,
        ],
    )
    def spmm(h_hbm, row_hbm, col_hbm, val_hbm, out_hbm,
             col_all, rowb, valb, rows0, rows1, acc, sem0, sem1):
        cid = lax.axis_index("c")
        sid = lax.axis_index("s")
        wid = cid * NS + sid
        base0 = wid * epw
        rows = (rows0, rows1)
        sems = (sem0, sem1)

        pltpu.sync_copy(col_hbm.at[pl.ds(base0, epw)], col_all)

        def zero_buf(i, carry):
            for j in range(D // LANES):
                rows0[i, pl.ds(LANES * j, LANES)] = jnp.zeros((LANES,), jnp.float32)
            return carry
        lax.fori_loop(0, CHUNK, zero_buf, 0)
        for b in range(rows_pt // CHUNK):
            pltpu.sync_copy(rows0, acc.at[pl.ds(sid * rows_pt + b * CHUNK, CHUNK)])
        rem = rows_pt % CHUNK
        if rem:
            pltpu.sync_copy(
                rows0.at[pl.ds(0, rem)],
                acc.at[pl.ds(sid * rows_pt + (rows_pt // CHUNK) * CHUNK, rem)])
        if tail:
            @pl.when(sid == NS - 1)
            def _zero_tail():
                pltpu.sync_copy(rows0.at[pl.ds(0, tail)],
                                acc.at[pl.ds(NS * rows_pt, tail)])
        plsc.subcore_barrier()

        def issue(ci, b):
            base = base0 + ci * CHUNK
            pltpu.async_copy(row_hbm.at[pl.ds(base, CHUNK)], rowb.at[b], sems[b])
            pltpu.async_copy(val_hbm.at[pl.ds(base, CHUNK)], valb.at[b], sems[b])
            pltpu.async_copy(h_hbm.at[col_all.at[pl.ds(ci * CHUNK, CHUNK)]],
                             rows[b], sems[b])

        def drain(b):
            pltpu.make_async_copy(row_hbm.at[pl.ds(0, CHUNK)], rowb.at[b], sems[b]).wait()
            pltpu.make_async_copy(val_hbm.at[pl.ds(0, CHUNK)], valb.at[b], sems[b]).wait()
            pltpu.make_async_copy(h_hbm.at[pl.ds(0, CHUNK)], rows[b], sems[b]).wait()

        def process(b):
            def scale(g, c2):
                val16 = valb[b, pl.ds(LANES * g, LANES)]
                for i in range(LANES):
                    s = val16[i]
                    e = LANES * g + i
                    for j in range(D // LANES):
                        sl = pl.ds(LANES * j, LANES)
                        rows[b][e, sl] = rows[b][e, sl] * s
                return c2
            lax.fori_loop(0, CHUNK // LANES, scale, 0)
            pltpu.sync_copy(rows[b], acc.at[rowb.at[b]], add=True)

        issue(0, 0)
        issue(1, 1)

        def pair_body(p, carry):
            ci0 = 2 * p
            drain(0)
            process(0)
            issue(ci0 + 2, 0)
            drain(1)
            process(1)

            @pl.when(ci0 + 3 < nchunk)
            def _issue_next():
                issue(ci0 + 3, 1)
            return carry

        lax.fori_loop(0, npair, pair_body, 0)
        drain(0)
        process(0)
        plsc.subcore_barrier()

        pltpu.sync_copy(acc.at[pl.ds(sid * rows_pt, rows_pt)],
                        out_hbm.at[cid].at[pl.ds(sid * rows_pt, rows_pt)])
        if tail:
            @pl.when(sid == NS - 1)
            def _write_tail():
                pltpu.sync_copy(acc.at[pl.ds(NS * rows_pt, tail)],
                                out_hbm.at[cid].at[pl.ds(NS * rows_pt, tail)])

    return spmm


def _tc_combine(p0, p1, W):
    N, D = p0.shape
    DO = W.shape[1]
    BM = 2000
    assert N % BM == 0

    def body(p0_ref, p1_ref, w_ref, o_ref):
        x = p0_ref[...] + p1_ref[...]
        o_ref[...] = jnp.dot(x, w_ref[...], preferred_element_type=jnp.float32)

    return pl.pallas_call(
        body,
        grid=(N // BM,),
        in_specs=[
            pl.BlockSpec((BM, D), lambda i: (i, 0)),
            pl.BlockSpec((BM, D), lambda i: (i, 0)),
            pl.BlockSpec((D, DO), lambda i: (0, 0)),
        ],
        out_specs=pl.BlockSpec((BM, DO), lambda i: (i, 0)),
        out_shape=jax.ShapeDtypeStruct((N, DO), jnp.float32),
    )(p0, p1, W)


def kernel(H, row, col, val, W):
    N, D = H.shape
    E = row.shape[0]
    partials = _sc_spmm(N, E, D)(H, row, col, val)
    return _tc_combine(partials[0], partials[1], W)
